# Initial kernel scaffold; baseline (speedup 1.0000x reference)
import jax
import jax.numpy as jnp
from jax.experimental import pallas as pl


def kernel(pos, y, batch, params):
    raise NotImplementedError("write your pallas kernel here")


# speedup vs baseline: 8.9335x; 8.9335x over previous
import functools

import jax
import jax.numpy as jnp
from jax import lax
from jax.experimental import pallas as pl
from jax.experimental.pallas import tpu as pltpu
from jax.experimental.pallas import tpu_sc as plsc

B = 4
N = 2048
K = 16
NSAMP = 500
_NW = 32


def _knn_body(xb_ref, xf_ref, xt_ref, idx_ref, *, rb):
    b = pl.program_id(0)
    j = pl.program_id(1)
    xb = xb_ref[0]
    xf = xf_ref[0]
    xt = xt_ref[0]
    e = lax.dot_general(xb, xt, (((1,), (0,)), ((), ())),
                        preferred_element_type=jnp.float32)
    sqb = jnp.sum(xb * xb, axis=-1)
    sqf = jnp.sum(xf * xf, axis=-1)
    d = sqb[:, None] + sqf[None, :] - 2.0 * e
    rows = j * rb + lax.broadcasted_iota(jnp.int32, (rb, N), 0)
    cols = lax.broadcasted_iota(jnp.int32, (rb, N), 1)
    d = jnp.where(rows == cols, jnp.inf, d)
    lanes = lax.broadcasted_iota(jnp.int32, (rb, K), 1)
    acc = jnp.zeros((rb, K), jnp.int32)
    for t in range(K):
        m = jnp.min(d, axis=1)
        hit = d == m[:, None]
        am = jnp.min(jnp.where(hit, cols, N), axis=1)
        acc = acc + jnp.where(lanes == t, am[:, None], 0)
        d = jnp.where(cols == am[:, None], jnp.inf, d)
    idx_ref[0] = acc + b * N


def _knn(x, xt, d_feat):
    rb = 256
    return pl.pallas_call(
        functools.partial(_knn_body, rb=rb),
        grid=(B, N // rb),
        in_specs=[
            pl.BlockSpec((1, rb, d_feat), lambda b, j: (b, j, 0)),
            pl.BlockSpec((1, N, d_feat), lambda b, j: (b, 0, 0)),
            pl.BlockSpec((1, d_feat, N), lambda b, j: (b, 0, 0)),
        ],
        out_specs=pl.BlockSpec((1, rb, K), lambda b, j: (b, j, 0)),
        out_shape=jax.ShapeDtypeStruct((B, N, K), jnp.int32),
    )(x, x, xt)


def _gather_rows(table, idx, chunk):
    e_total, d_feat = idx.shape[0], table.shape[1]
    epw = e_total // _NW
    nch = epw // chunk
    mesh = plsc.VectorSubcoreMesh(core_axis_name="c", subcore_axis_name="s")

    @functools.partial(
        pl.kernel, mesh=mesh,
        out_type=jax.ShapeDtypeStruct((e_total, d_feat), jnp.float32),
        compiler_params=pltpu.CompilerParams(use_tc_tiling_on_sc=False),
        scratch_types=[
            pltpu.VMEM((chunk,), jnp.int32),
            pltpu.VMEM((chunk, d_feat), jnp.float32),
            pltpu.SemaphoreType.DMA,
        ],
    )
    def gk(tab, idxh, outh, idx_v, buf, sem):
        wid = lax.axis_index("s") * 2 + lax.axis_index("c")
        base = wid * epw

        def body(c, carry):
            off = base + c * chunk
            pltpu.sync_copy(idxh.at[pl.ds(off, chunk)], idx_v)
            pltpu.async_copy(tab.at[idx_v], buf, sem).wait()
            pltpu.sync_copy(buf, outh.at[pl.ds(off, chunk)])
            return carry

        lax.fori_loop(0, nch, body, 0)

    return gk(table, idx)


def _final_gather_sc(tp, ty, t1, t2, t3, idx):
    epw = idx.shape[0] // _NW
    mesh = plsc.VectorSubcoreMesh(core_axis_name="c", subcore_axis_name="s")
    tabs = (tp, ty, t1, t2, t3)

    @functools.partial(
        pl.kernel, mesh=mesh,
        out_type=[jax.ShapeDtypeStruct((idx.shape[0], t.shape[1]), jnp.float32)
                  for t in tabs],
        compiler_params=pltpu.CompilerParams(use_tc_tiling_on_sc=False),
        scratch_types=[pltpu.VMEM((epw,), jnp.int32)] +
                      [pltpu.VMEM((epw, t.shape[1]), jnp.float32) for t in tabs] +
                      [pltpu.SemaphoreType.DMA],
    )
    def gk(h0, h1, h2, h3, h4, idxh, o0, o1, o2, o3, o4,
           idx_v, b0, b1, b2, b3, b4, sem):
        wid = lax.axis_index("s") * 2 + lax.axis_index("c")
        base = wid * epw
        pltpu.sync_copy(idxh.at[pl.ds(base, epw)], idx_v)
        for th, bh, oh in ((h0, b0, o0), (h1, b1, o1), (h2, b2, o2),
                           (h3, b3, o3), (h4, b4, o4)):
            pltpu.async_copy(th.at[idx_v], bh, sem).wait()
            pltpu.sync_copy(bh, oh.at[pl.ds(base, epw)])

    return gk(*tabs, idx)


def _edge_body(x_ref, g_ref, w1_ref, b1_ref, w2_ref, b2_ref, w3_ref, b3_ref,
               o_ref, *, rb, d_feat):
    xi = x_ref[0]
    g = g_ref[0][:, :, :d_feat]
    xi3 = jnp.broadcast_to(xi[:, None, :], (rb, K, d_feat))
    ef = jnp.concatenate([xi3, g - xi3], axis=-1).reshape(rb * K, 2 * d_feat)
    h = jax.nn.relu(jnp.dot(ef, w1_ref[...]) + b1_ref[...])
    h = jax.nn.relu(jnp.dot(h, w2_ref[...]) + b2_ref[...])
    h = jax.nn.relu(jnp.dot(h, w3_ref[...]) + b3_ref[...])
    o_ref[0] = jnp.max(h.reshape(rb, K, h.shape[-1]), axis=1)


def _edge(x, g, layers, d_feat, rb):
    (w1, b1), (w2, b2), (w3, b3) = layers
    d_tab = g.shape[-1]
    out_d = w3.shape[1]
    wspec = lambda w: pl.BlockSpec(w.shape, lambda b, j: tuple(0 for _ in w.shape))
    return pl.pallas_call(
        functools.partial(_edge_body, rb=rb, d_feat=d_feat),
        grid=(B, N // rb),
        in_specs=[
            pl.BlockSpec((1, rb, d_feat), lambda b, j: (b, j, 0)),
            pl.BlockSpec((1, rb, K, d_tab), lambda b, j: (b, j, 0, 0)),
            wspec(w1), wspec(b1), wspec(w2), wspec(b2), wspec(w3), wspec(b3),
        ],
        out_specs=pl.BlockSpec((1, rb, out_d), lambda b, j: (b, j, 0)),
        out_shape=jax.ShapeDtypeStruct((B, N, out_d), jnp.float32),
    )(x, g, w1, b1, w2, b2, w3, b3)


def _shared_body(x1_ref, x2_ref, x3_ref, ws1_ref, bs1_ref, ws2_ref, bs2_ref,
                 wc_ref, bc_ref, probs_ref, gemb_ref):
    j = pl.program_id(1)
    xcat = jnp.concatenate([x1_ref[0], x2_ref[0], x3_ref[0]], axis=-1)
    h = jax.nn.relu(jnp.dot(xcat, ws1_ref[...]) + bs1_ref[...])
    sh = jax.nn.relu(jnp.dot(h, ws2_ref[...]) + bs2_ref[...])
    logits = jnp.dot(sh, wc_ref[...]) + bc_ref[...]
    probs_ref[0] = jax.nn.sigmoid(logits)
    gm = jnp.max(sh, axis=0)

    @pl.when(j == 0)
    def _():
        gemb_ref[0, 0] = gm

    @pl.when(j != 0)
    def _():
        gemb_ref[0, 0] = jnp.maximum(gemb_ref[0, 0], gm)


def _shared(x1, x2, x3, ws1, bs1, ws2, bs2, wc, bc):
    rb = 512
    wspec = lambda w: pl.BlockSpec(w.shape, lambda b, j: tuple(0 for _ in w.shape))
    return pl.pallas_call(
        _shared_body,
        grid=(B, N // rb),
        in_specs=[
            pl.BlockSpec((1, rb, 32), lambda b, j: (b, j, 0)),
            pl.BlockSpec((1, rb, 128), lambda b, j: (b, j, 0)),
            pl.BlockSpec((1, rb, 512), lambda b, j: (b, j, 0)),
            wspec(ws1), wspec(bs1), wspec(ws2), wspec(bs2), wspec(wc), wspec(bc),
        ],
        out_specs=[
            pl.BlockSpec((1, rb, 1), lambda b, j: (b, j, 0)),
            pl.BlockSpec((1, 1, 128), lambda b, j: (b, 0, 0)),
        ],
        out_shape=[
            jax.ShapeDtypeStruct((B, N, 1), jnp.float32),
            jax.ShapeDtypeStruct((B, 1, 128), jnp.float32),
        ],
    )(x1, x2, x3, ws1, bs1, ws2, bs2, wc, bc)


def _grasp_body(l1_ref, l2_ref, l3_ref, rep_ref, ap_ref, w1_ref, b1_ref,
                w2_ref, b2_ref, o_ref):
    gf = jnp.concatenate(
        [l1_ref[...], l2_ref[...], l3_ref[...], rep_ref[...], ap_ref[...]],
        axis=-1)
    h = jax.nn.relu(jnp.dot(gf, w1_ref[...]) + b1_ref[...])
    o_ref[...] = jnp.dot(h, w2_ref[...]) + b2_ref[...]


def _grasp(l1, l2, l3, rep, ap, w1, b1, w2, b2):
    m = l1.shape[0]
    return pl.pallas_call(
        _grasp_body,
        out_shape=jax.ShapeDtypeStruct((m, 16), jnp.float32),
    )(l1, l2, l3, rep, ap, w1, b1, w2, b2)


def kernel(pos, y, batch, params):
    p = params
    r2 = lambda b: b.reshape(1, -1)

    x0 = pos.reshape(B, N, 3)
    pos_pad = jnp.pad(pos, ((0, 0), (0, 13)))

    idx1 = _knn(x0, jnp.swapaxes(x0, 1, 2), 3)
    g1 = _gather_rows(pos_pad, idx1.reshape(-1), 2048)
    c1 = [(w, r2(b)) for w, b in p['conv1']]
    x1 = _edge(x0, g1.reshape(B, N, K, 16), c1, 3, 256)

    idx2 = _knn(x1, jnp.swapaxes(x1, 1, 2), 32)
    g2 = _gather_rows(x1.reshape(B * N, 32), idx2.reshape(-1), 1024)
    c2 = [(w, r2(b)) for w, b in p['conv2']]
    x2 = _edge(x1, g2.reshape(B, N, K, 32), c2, 32, 256)

    idx3 = _knn(x2, jnp.swapaxes(x2, 1, 2), 128)
    g3 = _gather_rows(x2.reshape(B * N, 128), idx3.reshape(-1), 512)
    c3 = [(w, r2(b)) for w, b in p['conv3']]
    x3 = _edge(x2, g3.reshape(B, N, K, 128), c3, 128, 256)

    (ws1, bs1), (ws2, bs2) = p['shared']
    wc, bc = p['cls']
    probs3, gemb3 = _shared(x1, x2, x3, ws1, r2(bs1), ws2, r2(bs2), wc, r2(bc))
    gemb = gemb3[:, 0, :]
    probs = probs3[..., 0]
    cls_out = probs.reshape(-1)

    keys = jax.random.split(jax.random.key(42), B)
    idx_s = jnp.stack([
        jax.random.choice(keys[i], N, shape=(NSAMP,), replace=True,
                          p=probs[i] / jnp.sum(probs[i]))
        for i in range(B)
    ])
    gidx = (idx_s.astype(jnp.int32)
            + (jnp.arange(B, dtype=jnp.int32) * N)[:, None]).reshape(-1)
    gidx = jnp.concatenate(
        [gidx, jnp.zeros((2048 - B * NSAMP,), jnp.int32)])

    apg, ggtg, l1g, l2g, l3g = _final_gather_sc(
        pos_pad, y, x1.reshape(B * N, 32), x2.reshape(B * N, 128),
        x3.reshape(B * N, 512), gidx)
    m = B * NSAMP
    ap = apg[:m, :3]
    ggt = ggtg[:m]

    rep = jnp.tile(gemb, (NSAMP, 1))
    (wg1, bg1), (wg2, bg2) = p['grasp']
    go = _grasp(l1g[:m], l2g[:m], l3g[:m], rep, ap,
                wg1, r2(bg1), wg2, r2(bg2))
    return cls_out, go, ap, ggt
